# Initial kernel scaffold; baseline (speedup 1.0000x reference)
import functools

import jax
import jax.numpy as jnp
from jax import lax
from jax.experimental import pallas as pl
from jax.experimental.pallas import tpu as pltpu
from jax.experimental.pallas import tpu_sc as plsc

H = 128
K = 256
D = 32
C = 4096
NC = 2
NS = 16
NW = NC * NS
H_PER_W = H // NW
CHUNK = 1024
NCHUNK = C // CHUNK
LANES = 16


def _sc_body(table_hbm, idx_hbm, out_hbm, idx_v, rows_v, sem):
    wid = lax.axis_index("s") * NC + lax.axis_index("c")
    h0 = wid * H_PER_W

    def do_h(hl, _):
        h = h0 + hl
        off = h * K

        def do_chunk(ci, _):
            c0 = ci * CHUNK
            pltpu.sync_copy(idx_hbm.at[h, pl.ds(c0, CHUNK)], idx_v)

            def add_off(i, _):
                sl = pl.ds(i * LANES, LANES)
                idx_v[sl] = idx_v[sl] + off
                return 0

            lax.fori_loop(0, CHUNK // LANES, add_off, 0, unroll=8)
            pltpu.async_copy(table_hbm.at[idx_v], rows_v, sem).wait()
            pltpu.sync_copy(rows_v, out_hbm.at[pl.ds(c0, CHUNK), h])
            return 0

        lax.fori_loop(0, NCHUNK, do_chunk, 0)
        return 0

    lax.fori_loop(0, H_PER_W, do_h, 0)


@jax.jit
def _rpq_gather(table, indices):
    mesh = plsc.VectorSubcoreMesh(core_axis_name="c", subcore_axis_name="s")
    k = functools.partial(
        pl.kernel,
        mesh=mesh,
        out_type=jax.ShapeDtypeStruct((C, H, D), jnp.float32),
        scratch_types=[
            pltpu.VMEM((CHUNK,), jnp.int32),
            pltpu.VMEM((CHUNK, D), jnp.float32),
            pltpu.SemaphoreType.DMA,
        ],
    )(_sc_body)
    return k(table, indices)


def kernel(codebooks, indices):
    table = codebooks.reshape(H * K, D)
    out = _rpq_gather(table, indices)
    return out.reshape(C, H * D)


# speedup vs baseline: 21.3815x; 21.3815x over previous
import functools

import jax
import jax.numpy as jnp
from jax import lax
from jax.experimental import pallas as pl
from jax.experimental.pallas import tpu as pltpu
from jax.experimental.pallas import tpu_sc as plsc

H = 128
K = 256
D = 32
C = 4096
NC = 2
NS = 16
NW = NC * NS
H_PER_W = H // NW
CHUNK = 1024
NCHUNK = C // CHUNK
LANES = 16


def _sc_body(table_hbm, idx_hbm, out_hbm, idx_v, rows_v, sem):
    wid = lax.axis_index("s") * NC + lax.axis_index("c")
    h0 = wid * H_PER_W

    def do_h(hl, _):
        h = h0 + hl
        off = h * K

        def do_chunk(ci, _):
            c0 = ci * CHUNK
            pltpu.sync_copy(idx_hbm.at[h, pl.ds(c0, CHUNK)], idx_v)

            def add_off(i, _):
                sl = pl.ds(i * LANES, LANES)
                idx_v[sl] = idx_v[sl] + off
                return 0

            lax.fori_loop(0, CHUNK // LANES, add_off, 0, unroll=8)
            pltpu.async_copy(table_hbm.at[idx_v], rows_v, sem).wait()
            pltpu.sync_copy(rows_v, out_hbm.at[pl.ds(c0, CHUNK), h])
            return 0

        lax.fori_loop(0, NCHUNK, do_chunk, 0)
        return 0

    lax.fori_loop(0, H_PER_W, do_h, 0)


@jax.jit
def _rpq_gather(table, indices):
    mesh = plsc.VectorSubcoreMesh(core_axis_name="c", subcore_axis_name="s")
    k = functools.partial(
        pl.kernel,
        mesh=mesh,
        out_type=jax.ShapeDtypeStruct((C, H, D), jnp.float32),
        scratch_types=[
            pltpu.VMEM((CHUNK,), jnp.int32),
            pltpu.VMEM((CHUNK, D), jnp.float32),
            pltpu.SemaphoreType.DMA,
        ],
        compiler_params=pltpu.CompilerParams(use_tc_tiling_on_sc=False),
    )(_sc_body)
    return k(table, indices)


def kernel(codebooks, indices):
    table = codebooks.reshape(H * K, D)
    out = _rpq_gather(table, indices)
    return out.reshape(C, H * D)
